# baseline (speedup 1.0000x reference)
import functools

import jax
import jax.numpy as jnp
from jax import lax
from jax.experimental import pallas as pl
from jax.experimental.pallas import tpu as pltpu
from jax.experimental.pallas import tpu_sc as plsc

_N = 10000
_E = 320000
_C = 128
_K = 27
_KC = _K * _C

_NC = 2
_NS = 16
_NW = _NC * _NS
_CHUNK = 64
_NCHUNKS = 160
_NPAIRS = 80
_EPW = _NCHUNKS * _CHUNK
_EPAD = _NW * _EPW
_NPAD = 10112
_ROWS_PER_TILE = _NPAD // _NS
_ACC_ROWS = _NPAD

_BN = 400


def _prep_body(src_ref, kid_ref, o_ref):
    o_ref[...] = kid_ref[...] * _N + src_ref[...]


def _prep(src, kid):
    s2 = src.reshape(_E // _C, _C)
    k2 = kid.reshape(_E // _C, _C)
    out = pl.pallas_call(
        _prep_body,
        out_shape=jax.ShapeDtypeStruct((_E // _C, _C), jnp.int32),
    )(s2, k2)
    return out.reshape(_E)


def _mm_body(x_ref, w_ref, o_ref):
    o_ref[0] = jnp.dot(x_ref[...], w_ref[0],
                       preferred_element_type=jnp.float32)


def _mm(x, w):
    return pl.pallas_call(
        _mm_body,
        grid=(_N // _BN, _K),
        in_specs=[pl.BlockSpec((_BN, _C), lambda i, k: (i, 0)),
                  pl.BlockSpec((1, _C, _C), lambda i, k: (k, 0, 0))],
        out_specs=pl.BlockSpec((1, _BN, _C), lambda i, k: (k, i, 0)),
        out_shape=jax.ShapeDtypeStruct((_K, _N, _C), jnp.float32),
    )(x, w)


def _mm_fused_body(p0_ref, p1_ref, w_ref, o_ref):
    h = jnp.maximum(p0_ref[...] + p1_ref[...], 0.0)
    o_ref[0] = jnp.dot(h, w_ref[0], preferred_element_type=jnp.float32)


def _mm_fused(p, w):
    return pl.pallas_call(
        _mm_fused_body,
        grid=(_N // _BN, _K),
        in_specs=[pl.BlockSpec((_BN, _C), lambda i, k: (i, 0)),
                  pl.BlockSpec((_BN, _C), lambda i, k: (i, 0)),
                  pl.BlockSpec((1, _C, _C), lambda i, k: (k, 0, 0))],
        out_specs=pl.BlockSpec((1, _BN, _C), lambda i, k: (k, i, 0)),
        out_shape=jax.ShapeDtypeStruct((_K, _N, _C), jnp.float32),
    )(p[0], p[1], w)


def _final_body(q0_ref, q1_ref, x_ref, o_ref):
    o_ref[...] = jnp.maximum(q0_ref[...] + q1_ref[...] + x_ref[...], 0.0)


def _final(q, x):
    bn = 2000
    return pl.pallas_call(
        _final_body,
        grid=(_N // bn,),
        in_specs=[pl.BlockSpec((bn, _C), lambda i: (i, 0)),
                  pl.BlockSpec((bn, _C), lambda i: (i, 0)),
                  pl.BlockSpec((bn, _C), lambda i: (i, 0))],
        out_specs=pl.BlockSpec((bn, _C), lambda i: (i, 0)),
        out_shape=jax.ShapeDtypeStruct((_N, _C), jnp.float32),
    )(q[0], q[1], x)


_mesh = plsc.VectorSubcoreMesh(core_axis_name="c", subcore_axis_name="s")


_NBUF = 2
_ISLOTS = 8


@functools.partial(
    pl.kernel,
    mesh=_mesh,
    out_type=jax.ShapeDtypeStruct((_NC, _NPAD, _C), jnp.float32),
    scratch_types=[
        pltpu.VMEM((_ISLOTS, 1, _CHUNK), jnp.int32),
        pltpu.VMEM((_NPAIRS, 2 * _CHUNK), jnp.int32),
        pltpu.VMEM((2 * _CHUNK, _C), jnp.float32),
        pltpu.VMEM((2 * _CHUNK, _C), jnp.float32),
        pltpu.VMEM_SHARED((_ACC_ROWS, _C), jnp.float32),
        pltpu.SemaphoreType.DMA((4,)),
        pltpu.SemaphoreType.DMA((_ISLOTS,)),
    ],
)
def _gather_scatter(gidx_hbm, didx_hbm, table_hbm, zeros_hbm, out_hbm,
                    gidx_r, didx_v, pair0, pair1, acc_sh, gsem, isem):
    pair_bufs = (pair0, pair1)
    cid = lax.axis_index("c")
    sid = lax.axis_index("s")
    wid = sid * _NC + cid
    r0 = sid * _ROWS_PER_TILE

    pltpu.sync_copy(zeros_hbm.at[pl.ds(r0, _ROWS_PER_TILE)],
                    acc_sh.at[pl.ds(r0, _ROWS_PER_TILE)])
    pltpu.sync_copy(didx_hbm.at[wid], didx_v)

    def i_start(i, slot):
        pltpu.async_copy(gidx_hbm.at[wid * _NCHUNKS + i], gidx_r.at[slot],
                         isem.at[slot])

    def i_wait(i, slot):
        pltpu.make_async_copy(gidx_hbm.at[wid * _NCHUNKS + i], gidx_r.at[slot],
                              isem.at[slot]).wait()

    def g_start(i, slot, q, h):
        pltpu.async_copy(table_hbm.at[gidx_r.at[slot, 0]],
                         pair_bufs[q].at[pl.ds(h * _CHUNK, _CHUNK)],
                         gsem.at[2 * q + h])

    def g_wait(i, slot, q, h):
        pltpu.make_async_copy(table_hbm.at[gidx_r.at[slot, 0]],
                              pair_bufs[q].at[pl.ds(h * _CHUNK, _CHUNK)],
                              gsem.at[2 * q + h]).wait()

    plsc.subcore_barrier()

    for c in range(_ISLOTS):
        i_start(c, c)
    for c in range(4):
        i_wait(c, c)
        g_start(c, c, (c // 2) % 2, c % 2)

    def outer(step, carry):
        base = step * _ISLOTS
        for c in range(_ISLOTS):
            i = base + c
            q = (c // 2) % 2
            h = c % 2
            g_wait(i, c, q, h)

            @pl.when(i + _ISLOTS < _NCHUNKS)
            def _():
                i_start(i + _ISLOTS, c)

            if h == 1:
                pltpu.sync_copy(pair_bufs[q], acc_sh.at[didx_v.at[i // 2]],
                                add=True)
                for h2 in range(2):
                    inew = i - 1 + h2 + 4
                    cnew = (c - 1 + h2 + 4) % _ISLOTS

                    @pl.when(inew < _NCHUNKS)
                    def _():
                        i_wait(inew, cnew)
                        g_start(inew, cnew, q, h2)
        return carry

    lax.fori_loop(0, _NCHUNKS // _ISLOTS, outer, 0)
    plsc.subcore_barrier()
    pltpu.sync_copy(acc_sh.at[pl.ds(r0, _ROWS_PER_TILE)],
                    out_hbm.at[cid, pl.ds(r0, _ROWS_PER_TILE)])


def kernel(x, edge_index, kernel_id, W1, W2):
    src = edge_index[0]
    dst = edge_index[1]

    gidx = _prep(src, kernel_id)
    pad = _EPAD - _E
    gidx_p = jnp.concatenate(
        [gidx, jnp.zeros((pad,), jnp.int32)]).reshape(_NW * _NCHUNKS, 1, _CHUNK)
    didx_p = jnp.concatenate(
        [dst, jnp.full((pad,), _N, jnp.int32)]).reshape(_NW, _NPAIRS, 2 * _CHUNK)
    zeros = jnp.zeros((_NPAD, _C), jnp.float32)

    y1 = _mm(x, W1).reshape(_N * _K, _C)
    p = _gather_scatter(gidx_p, didx_p, y1, zeros)
    y2 = _mm_fused(p, W2).reshape(_N * _K, _C)
    q = _gather_scatter(gidx_p, didx_p, y2, zeros)
    return _final(q, x)


# speedup vs baseline: 1.7474x; 1.7474x over previous
import functools

import jax
import jax.numpy as jnp
from jax import lax
from jax.experimental import pallas as pl
from jax.experimental.pallas import tpu as pltpu
from jax.experimental.pallas import tpu_sc as plsc

_N = 10000
_E = 320000
_C = 128
_K = 27
_KC = _K * _C

_NC = 2
_NS = 16
_NW = _NC * _NS
_CHUNK = 64
_NCHUNKS = 160
_NPAIRS = 80
_EPW = _NCHUNKS * _CHUNK
_EPAD = _NW * _EPW
_NPAD = 10112
_ROWS_PER_TILE = _NPAD // _NS
_ACC_ROWS = _NPAD

_BN = 400


def _prep_body(src_ref, kid_ref, o_ref):
    o_ref[...] = kid_ref[...] * _N + src_ref[...]


def _prep(src, kid):
    s2 = src.reshape(_E // _C, _C)
    k2 = kid.reshape(_E // _C, _C)
    out = pl.pallas_call(
        _prep_body,
        out_shape=jax.ShapeDtypeStruct((_E // _C, _C), jnp.int32),
    )(s2, k2)
    return out.reshape(_E)


def _mm_body(x_ref, w_ref, o_ref):
    for k in range(_K):
        o_ref[k] = jnp.dot(x_ref[...], w_ref[k],
                           preferred_element_type=jnp.float32)


def _mm(x, w):
    return pl.pallas_call(
        _mm_body,
        grid=(_N // _BN,),
        in_specs=[pl.BlockSpec((_BN, _C), lambda i: (i, 0)),
                  pl.BlockSpec((_K, _C, _C), lambda i: (0, 0, 0))],
        out_specs=pl.BlockSpec((_K, _BN, _C), lambda i: (0, i, 0)),
        out_shape=jax.ShapeDtypeStruct((_K, _N, _C), jnp.float32),
    )(x, w)


def _mm_fused_body(p0_ref, p1_ref, w_ref, o_ref):
    h = jnp.maximum(p0_ref[...] + p1_ref[...], 0.0)
    for k in range(_K):
        o_ref[k] = jnp.dot(h, w_ref[k], preferred_element_type=jnp.float32)


def _mm_fused(p, w):
    return pl.pallas_call(
        _mm_fused_body,
        grid=(_N // _BN,),
        in_specs=[pl.BlockSpec((_BN, _C), lambda i: (i, 0)),
                  pl.BlockSpec((_BN, _C), lambda i: (i, 0)),
                  pl.BlockSpec((_K, _C, _C), lambda i: (0, 0, 0))],
        out_specs=pl.BlockSpec((_K, _BN, _C), lambda i: (0, i, 0)),
        out_shape=jax.ShapeDtypeStruct((_K, _N, _C), jnp.float32),
    )(p[0], p[1], w)


def _final_body(q0_ref, q1_ref, x_ref, o_ref):
    o_ref[...] = jnp.maximum(q0_ref[...] + q1_ref[...] + x_ref[...], 0.0)


def _final(q, x):
    bn = 2000
    return pl.pallas_call(
        _final_body,
        grid=(_N // bn,),
        in_specs=[pl.BlockSpec((bn, _C), lambda i: (i, 0)),
                  pl.BlockSpec((bn, _C), lambda i: (i, 0)),
                  pl.BlockSpec((bn, _C), lambda i: (i, 0))],
        out_specs=pl.BlockSpec((bn, _C), lambda i: (i, 0)),
        out_shape=jax.ShapeDtypeStruct((_N, _C), jnp.float32),
    )(q[0], q[1], x)


_mesh = plsc.VectorSubcoreMesh(core_axis_name="c", subcore_axis_name="s")


_NBUF = 2
_ISLOTS = 8


@functools.partial(
    pl.kernel,
    mesh=_mesh,
    out_type=jax.ShapeDtypeStruct((_NC, _NPAD, _C), jnp.float32),
    scratch_types=[
        pltpu.VMEM((_ISLOTS, 1, _CHUNK), jnp.int32),
        pltpu.VMEM((_NPAIRS, 2 * _CHUNK), jnp.int32),
        pltpu.VMEM((2 * _CHUNK, _C), jnp.float32),
        pltpu.VMEM((2 * _CHUNK, _C), jnp.float32),
        pltpu.VMEM_SHARED((_ACC_ROWS, _C), jnp.float32),
        pltpu.SemaphoreType.DMA((4,)),
        pltpu.SemaphoreType.DMA((_ISLOTS,)),
    ],
)
def _gather_scatter(gidx_hbm, didx_hbm, table_hbm, zeros_hbm, out_hbm,
                    gidx_r, didx_v, pair0, pair1, acc_sh, gsem, isem):
    pair_bufs = (pair0, pair1)
    cid = lax.axis_index("c")
    sid = lax.axis_index("s")
    wid = sid * _NC + cid
    r0 = sid * _ROWS_PER_TILE

    pltpu.sync_copy(zeros_hbm.at[pl.ds(r0, _ROWS_PER_TILE)],
                    acc_sh.at[pl.ds(r0, _ROWS_PER_TILE)])
    pltpu.sync_copy(didx_hbm.at[wid], didx_v)

    def i_start(i, slot):
        pltpu.async_copy(gidx_hbm.at[wid * _NCHUNKS + i], gidx_r.at[slot],
                         isem.at[slot])

    def i_wait(i, slot):
        pltpu.make_async_copy(gidx_hbm.at[wid * _NCHUNKS + i], gidx_r.at[slot],
                              isem.at[slot]).wait()

    def g_start(i, slot, q, h):
        pltpu.async_copy(table_hbm.at[gidx_r.at[slot, 0]],
                         pair_bufs[q].at[pl.ds(h * _CHUNK, _CHUNK)],
                         gsem.at[2 * q + h])

    def g_wait(i, slot, q, h):
        pltpu.make_async_copy(table_hbm.at[gidx_r.at[slot, 0]],
                              pair_bufs[q].at[pl.ds(h * _CHUNK, _CHUNK)],
                              gsem.at[2 * q + h]).wait()

    plsc.subcore_barrier()

    for c in range(_ISLOTS):
        i_start(c, c)
    for c in range(4):
        i_wait(c, c)
        g_start(c, c, (c // 2) % 2, c % 2)

    def outer(step, carry):
        base = step * _ISLOTS
        for c in range(_ISLOTS):
            i = base + c
            q = (c // 2) % 2
            h = c % 2
            g_wait(i, c, q, h)

            @pl.when(i + _ISLOTS < _NCHUNKS)
            def _():
                i_start(i + _ISLOTS, c)

            if h == 1:
                pltpu.sync_copy(pair_bufs[q], acc_sh.at[didx_v.at[i // 2]],
                                add=True)
                for h2 in range(2):
                    inew = i - 1 + h2 + 4
                    cnew = (c - 1 + h2 + 4) % _ISLOTS

                    @pl.when(inew < _NCHUNKS)
                    def _():
                        i_wait(inew, cnew)
                        g_start(inew, cnew, q, h2)
        return carry

    lax.fori_loop(0, _NCHUNKS // _ISLOTS, outer, 0)
    plsc.subcore_barrier()
    pltpu.sync_copy(acc_sh.at[pl.ds(r0, _ROWS_PER_TILE)],
                    out_hbm.at[cid, pl.ds(r0, _ROWS_PER_TILE)])


def kernel(x, edge_index, kernel_id, W1, W2):
    src = edge_index[0]
    dst = edge_index[1]

    gidx = _prep(src, kernel_id)
    pad = _EPAD - _E
    gidx_p = jnp.concatenate(
        [gidx, jnp.zeros((pad,), jnp.int32)]).reshape(_NW * _NCHUNKS, 1, _CHUNK)
    didx_p = jnp.concatenate(
        [dst, jnp.full((pad,), _N, jnp.int32)]).reshape(_NW, _NPAIRS, 2 * _CHUNK)
    zeros = jnp.zeros((_NPAD, _C), jnp.float32)

    y1 = _mm(x, W1).reshape(_N * _K, _C)
    p = _gather_scatter(gidx_p, didx_p, y1, zeros)
    y2 = _mm_fused(p, W2).reshape(_N * _K, _C)
    q = _gather_scatter(gidx_p, didx_p, y2, zeros)
    return _final(q, x)
